# Initial kernel scaffold; baseline (speedup 1.0000x reference)
import jax
import jax.numpy as jnp
from jax.experimental import pallas as pl


def kernel(nodes, history_uv, history_r, feat_table, r_table, W_gv, b_gv, W1, b1):
    raise NotImplementedError("write your pallas kernel here")


# speedup vs baseline: 8.2566x; 8.2566x over previous
import functools

import jax
import jax.numpy as jnp
from jax import lax
from jax.experimental import pallas as pl
from jax.experimental.pallas import tpu as pltpu
from jax.experimental.pallas import tpu_sc as plsc

D = 128
L = 32

_NC = 2
_NS = 16
_NW = _NC * _NS


def _proj_kernel(feat_ref, a_ref, w1a_ref, p_ref, f1_ref):
    f = feat_ref[...]
    p_ref[...] = jnp.dot(f, a_ref[...], preferred_element_type=jnp.float32)
    f1_ref[...] = jnp.dot(f, w1a_ref[...], preferred_element_type=jnp.float32)


def _ctab_kernel(r_ref, bm_ref, bgv_ref, c_ref):
    c_ref[...] = (
        jnp.dot(r_ref[...], bm_ref[...], preferred_element_type=jnp.float32)
        + bgv_ref[...]
    )


def _combine_kernel(g_ref, r_ref, c_ref, s1_ref, w1b_ref, b1_ref, out_ref):
    g = g_ref[...]
    r = r_ref[0, 0, :]
    oh = (r[:, None] == lax.broadcasted_iota(jnp.int32, (r.shape[0], 8), 1))
    rc = jnp.dot(oh.astype(jnp.float32), c_ref[...],
                 preferred_element_type=jnp.float32)
    h = jnp.maximum(g + rc, 0.0)
    neigh = jnp.sum(h.reshape(-1, L, D), axis=1) * (1.0 / L)
    comb = (s1_ref[...]
            + jnp.dot(neigh, w1b_ref[...], preferred_element_type=jnp.float32)
            + b1_ref[...])
    out_ref[...] = jnp.maximum(comb, 0.0)


def _sc_gather_body(p_hbm, f1_hbm, uv_hbm, nodes_hbm, g_out, s_out,
                    idx_v, rows_v, sem):
    wid = lax.axis_index("s") * _NC + lax.axis_index("c")
    nch = uv_hbm.shape[0] // (_NW * 4)

    def body(t, carry):
        cbase = (wid * nch + t) * 4
        pltpu.sync_copy(uv_hbm.at[pl.ds(cbase, 4)], idx_v)
        descs = [
            pltpu.async_copy(p_hbm.at[idx_v.at[j]],
                             rows_v.at[pl.ds(j * 128, 128)], sem)
            for j in range(4)
        ]
        for d in descs:
            d.wait()
        pltpu.sync_copy(rows_v, g_out.at[pl.ds(cbase * 128, 512)])
        return carry

    lax.fori_loop(0, nch, body, 0)

    pltpu.sync_copy(nodes_hbm.at[pl.ds(wid * 4, 4)], idx_v)
    descs = [
        pltpu.async_copy(f1_hbm.at[idx_v.at[j]],
                         rows_v.at[pl.ds(j * 128, 128)], sem)
        for j in range(4)
    ]
    for d in descs:
        d.wait()
    pltpu.sync_copy(rows_v, s_out.at[pl.ds(wid * 512, 512)])


def kernel(nodes, history_uv, history_r, feat_table, r_table, W_gv, b_gv, W1, b1):
    B = nodes.shape[0]
    V = feat_table.shape[0]
    BL = B * L

    nodes_i = nodes.astype(jnp.int32).reshape(B // 128, 128)
    uv_i = history_uv.astype(jnp.int32).reshape(BL // 128, 128)
    r3 = history_r.astype(jnp.int32).reshape(B // 128, 1, 128 * L)

    A = W_gv[:D]
    Bm = W_gv[D:]
    W1a = W1[:D]
    W1b = W1[D:]
    r_pad = jnp.pad(r_table, ((0, 8 - r_table.shape[0]), (0, 0)))

    rb = 10000
    P, F1 = pl.pallas_call(
        _proj_kernel,
        grid=(V // rb,),
        in_specs=[
            pl.BlockSpec((rb, D), lambda i: (i, 0)),
            pl.BlockSpec((D, D), lambda i: (0, 0)),
            pl.BlockSpec((D, D), lambda i: (0, 0)),
        ],
        out_specs=[
            pl.BlockSpec((rb, D), lambda i: (i, 0)),
            pl.BlockSpec((rb, D), lambda i: (i, 0)),
        ],
        out_shape=[jax.ShapeDtypeStruct((V, D), jnp.float32)] * 2,
    )(feat_table, A, W1a)

    C = pl.pallas_call(
        _ctab_kernel,
        out_shape=jax.ShapeDtypeStruct((8, D), jnp.float32),
    )(r_pad, Bm, b_gv.reshape(1, D))

    mesh = plsc.VectorSubcoreMesh(core_axis_name="c", subcore_axis_name="s")
    sc_gather = functools.partial(
        pl.kernel,
        mesh=mesh,
        out_type=(
            jax.ShapeDtypeStruct((BL, D), jnp.float32),
            jax.ShapeDtypeStruct((B, D), jnp.float32),
        ),
        scratch_types=[
            pltpu.VMEM((4, 128), jnp.int32),
            pltpu.VMEM((512, D), jnp.float32),
            pltpu.SemaphoreType.DMA,
        ],
    )(_sc_gather_body)
    G, S1 = sc_gather(P, F1, uv_i, nodes_i)

    rbl = 128 * L
    out = pl.pallas_call(
        _combine_kernel,
        grid=(B // 128,),
        in_specs=[
            pl.BlockSpec((rbl, D), lambda i: (i, 0)),
            pl.BlockSpec((1, 1, rbl), lambda i: (i, 0, 0)),
            pl.BlockSpec((8, D), lambda i: (0, 0)),
            pl.BlockSpec((128, D), lambda i: (i, 0)),
            pl.BlockSpec((D, D), lambda i: (0, 0)),
            pl.BlockSpec((1, D), lambda i: (0, 0)),
        ],
        out_specs=pl.BlockSpec((128, D), lambda i: (i, 0)),
        out_shape=jax.ShapeDtypeStruct((B, D), jnp.float32),
    )(G, r3, C, S1, W1b, b1.reshape(1, D))
    return out
